# baseline (speedup 1.0000x reference)
import functools

import jax
import jax.numpy as jnp
import numpy as np
from jax import lax
from jax.experimental import pallas as pl
from jax.experimental.pallas import tpu as pltpu
from jax.experimental.pallas import tpu_sc as plsc

_VOCAB = 1000000
_D = 64
_T = 200
_SCALE = 8.0

_NC = 2
_NS = 16
_NW = _NC * _NS
_CHUNK = 128


def _pos_encoding_ext() -> np.ndarray:
    depth = _D / 2
    positions = np.arange(_T)[:, np.newaxis]
    depths = np.arange(depth)[np.newaxis, :] / depth
    angle_rads = positions * (1.0 / 10000**depths)
    pos = np.concatenate([np.sin(angle_rads), np.cos(angle_rads)], axis=-1)
    pos = pos.astype(np.float32)
    return np.concatenate([pos, pos[:_CHUNK]], axis=0)


def _sc_kernel(n_rows: int):
    rows_per_w = n_rows // _NW
    n_chunks = rows_per_w // _CHUNK
    mesh = plsc.VectorSubcoreMesh(
        core_axis_name="c", subcore_axis_name="s", num_cores=_NC,
        num_subcores=_NS)

    @functools.partial(
        pl.kernel,
        out_type=jax.ShapeDtypeStruct((n_rows, _D), jnp.float32),
        mesh=mesh,
        scratch_types=[
            pltpu.VMEM((_CHUNK,), jnp.int32),
            pltpu.VMEM((_CHUNK, _D), jnp.float32),
            pltpu.VMEM((_T + _CHUNK, _D), jnp.float32),
            pltpu.SemaphoreType.DMA,
        ],
        compiler_params=pltpu.CompilerParams(use_tc_tiling_on_sc=False),
    )
    def body(x_hbm, table_hbm, pos_hbm, out_hbm, idx_v, rows_v, pos_v, sem):
        wid = lax.axis_index("s") * _NC + lax.axis_index("c")
        pltpu.sync_copy(pos_hbm, pos_v)
        w_base = wid * rows_per_w

        def chunk_body(c, carry):
            base = w_base + c * _CHUNK
            pltpu.sync_copy(x_hbm.at[pl.ds(base, _CHUNK)], idx_v)
            pltpu.async_copy(table_hbm.at[idx_v], rows_v, sem).wait()
            p0 = lax.rem(c * _CHUNK, _T)

            def row_body(i, rcarry):
                for j in range(_D // 16):
                    v = rows_v[i, pl.ds(j * 16, 16)]
                    p = pos_v[p0 + i, pl.ds(j * 16, 16)]
                    rows_v[i, pl.ds(j * 16, 16)] = v * _SCALE + p
                return rcarry

            lax.fori_loop(0, _CHUNK, row_body, 0, unroll=2)
            pltpu.sync_copy(rows_v, out_hbm.at[pl.ds(base, _CHUNK)])
            return carry

        lax.fori_loop(0, n_chunks, chunk_body, 0)

    return body


@jax.jit
def kernel(x, embedding_table):
    b, t = x.shape
    x_flat = x.reshape(-1).astype(jnp.int32)
    pos_ext = jnp.asarray(_pos_encoding_ext())
    out = _sc_kernel(b * t)(x_flat, embedding_table, pos_ext)
    return out.reshape(b, t, _D)


# speedup vs baseline: 1.1660x; 1.1660x over previous
import functools

import jax
import jax.numpy as jnp
import numpy as np
from jax import lax
from jax.experimental import pallas as pl
from jax.experimental.pallas import tpu as pltpu
from jax.experimental.pallas import tpu_sc as plsc

_VOCAB = 1000000
_D = 64
_T = 200
_SCALE = 8.0

_NC = 2
_NS = 16
_NW = _NC * _NS
_CHUNK = 128
_NBUF = 4


def _pos_encoding_ext() -> np.ndarray:
    depth = _D / 2
    positions = np.arange(_T)[:, np.newaxis]
    depths = np.arange(depth)[np.newaxis, :] / depth
    angle_rads = positions * (1.0 / 10000**depths)
    pos = np.concatenate([np.sin(angle_rads), np.cos(angle_rads)], axis=-1)
    pos = pos.astype(np.float32)
    return np.concatenate([pos, pos[:_CHUNK]], axis=0)


def _sc_kernel(n_rows: int):
    rows_per_w = n_rows // _NW
    n_chunks = rows_per_w // _CHUNK
    n_groups = n_chunks // _NBUF
    mesh = plsc.VectorSubcoreMesh(
        core_axis_name="c", subcore_axis_name="s", num_cores=_NC,
        num_subcores=_NS)

    @functools.partial(
        pl.kernel,
        out_type=jax.ShapeDtypeStruct((n_rows, _D), jnp.float32),
        mesh=mesh,
        scratch_types=[
            pltpu.VMEM((n_chunks, _CHUNK), jnp.int32),
            pltpu.VMEM((_NBUF, _CHUNK, _D), jnp.float32),
            pltpu.VMEM((_NBUF, _CHUNK, _D), jnp.float32),
            pltpu.VMEM((_T + _CHUNK, _D), jnp.float32),
        ] + [pltpu.SemaphoreType.DMA] * (2 * _NBUF),
        compiler_params=pltpu.CompilerParams(use_tc_tiling_on_sc=False),
    )
    def body(x_hbm, table_hbm, pos_hbm, out_hbm, idx_slab, rows_in, rows_out,
             pos_v, *sems):
        gsem = sems[:_NBUF]
        wsem = sems[_NBUF:]
        wid = lax.axis_index("s") * _NC + lax.axis_index("c")
        pltpu.sync_copy(pos_hbm, pos_v)
        pltpu.sync_copy(x_hbm.at[pl.ds(wid * n_chunks, n_chunks)], idx_slab)
        w_base = wid * rows_per_w

        def start_gather(g, b):
            pltpu.async_copy(
                table_hbm.at[idx_slab.at[g]], rows_in.at[b], gsem[b])

        for b in range(_NBUF):
            start_gather(b, b)

        def group_body(gi, carry):
            g0 = gi * _NBUF
            for b in range(_NBUF):
                g = g0 + b
                base = w_base + g * _CHUNK

                @pl.when(gi > 0)
                def _wait_wb():
                    pltpu.make_async_copy(
                        rows_out.at[b], out_hbm.at[pl.ds(0, _CHUNK)],
                        wsem[b]).wait()

                pltpu.make_async_copy(
                    table_hbm.at[idx_slab.at[g]], rows_in.at[b],
                    gsem[b]).wait()

                p0 = lax.rem(g * _CHUNK, _T)

                def row_body(i, rcarry):
                    for j in range(_D // 16):
                        v = rows_in[b, i, pl.ds(j * 16, 16)]
                        p = pos_v[p0 + i, pl.ds(j * 16, 16)]
                        rows_out[b, i, pl.ds(j * 16, 16)] = v * _SCALE + p
                    return rcarry

                lax.fori_loop(0, _CHUNK, row_body, 0, unroll=2)
                start_gather(jnp.minimum(g + _NBUF, n_chunks - 1), b)
                pltpu.async_copy(
                    rows_out.at[b], out_hbm.at[pl.ds(base, _CHUNK)], wsem[b])
            return carry

        lax.fori_loop(0, n_groups, group_body, 0)

        for b in range(_NBUF):
            pltpu.make_async_copy(
                table_hbm.at[idx_slab.at[0]], rows_in.at[b], gsem[b]).wait()
            pltpu.make_async_copy(
                rows_out.at[b], out_hbm.at[pl.ds(0, _CHUNK)], wsem[b]).wait()

    return body


@jax.jit
def kernel(x, embedding_table):
    b, t = x.shape
    x_flat = x.reshape(-1, _CHUNK).astype(jnp.int32)
    pos_ext = jnp.asarray(_pos_encoding_ext())
    out = _sc_kernel(b * t)(x_flat, embedding_table, pos_ext)
    return out.reshape(b, t, _D)


# speedup vs baseline: 1.4740x; 1.2642x over previous
import functools

import jax
import jax.numpy as jnp
import numpy as np
from jax import lax
from jax.experimental import pallas as pl
from jax.experimental.pallas import tpu as pltpu
from jax.experimental.pallas import tpu_sc as plsc

_VOCAB = 1000000
_D = 64
_T = 200
_B = 4096
_SCALE = 8.0

_NC = 2
_NS = 16
_NW = _NC * _NS
_BSUB = 128
_NBB = _B // _BSUB
_NTB = _T // 8
_NIN = 4
_NOUT = 2


def _pos_encoding() -> np.ndarray:
    depth = _D / 2
    positions = np.arange(_T)[:, np.newaxis]
    depths = np.arange(depth)[np.newaxis, :] / depth
    angle_rads = positions * (1.0 / 10000**depths)
    pos = np.concatenate([np.sin(angle_rads), np.cos(angle_rads)], axis=-1)
    return pos.astype(np.float32)


def _sc_kernel():
    mesh = plsc.VectorSubcoreMesh(
        core_axis_name="c", subcore_axis_name="s", num_cores=_NC,
        num_subcores=_NS)

    @functools.partial(
        pl.kernel,
        out_type=jax.ShapeDtypeStruct((_T * (_D // 8) * _NBB, 8, _BSUB),
                                      jnp.float32),
        mesh=mesh,
        scratch_types=[
            pltpu.VMEM((_NTB, 8, _BSUB), jnp.int32),
            pltpu.VMEM((_NIN, _BSUB, _D), jnp.float32),
            pltpu.VMEM((_NOUT, _D, _BSUB), jnp.float32),
            pltpu.VMEM((_T, _D), jnp.float32),
        ] + [pltpu.SemaphoreType.DMA] * (_NIN + _NOUT),
        compiler_params=pltpu.CompilerParams(
            use_tc_tiling_on_sc=False, needs_layout_passes=False),
    )
    def body(x_hbm, table_hbm, pos_hbm, out_hbm, idx_slab, rows_in, rows_out,
             pos_v, *sems):
        gsem = sems[:_NIN]
        wsem = sems[_NIN:]
        wid = lax.axis_index("s") * _NC + lax.axis_index("c")
        pltpu.sync_copy(pos_hbm, pos_v)
        for tb in range(_NTB):
            pltpu.sync_copy(x_hbm.at[tb * _NBB + wid], idx_slab.at[tb])

        def start_gather(t, b):
            pltpu.async_copy(
                table_hbm.at[idx_slab.at[t // 8, t % 8]], rows_in.at[b],
                gsem[b])

        for b in range(_NIN):
            start_gather(b, b)

        lane = lax.iota(jnp.int32, 16)
        rowvec = [lane + 16 * s for s in range(_BSUB // 16)]

        def group_body(gi, carry):
            t0 = gi * _NIN
            for b in range(_NIN):
                t = t0 + b
                so = b % _NOUT

                def _wait_wb():
                    for k in range(_D // 8):
                        pltpu.make_async_copy(
                            rows_out.at[so, pl.ds(8 * k, 8)], out_hbm.at[k],
                            wsem[so]).wait()

                if b >= _NOUT:
                    _wait_wb()
                else:
                    pl.when(gi > 0)(_wait_wb)

                pltpu.make_async_copy(
                    table_hbm.at[idx_slab.at[t // 8, t % 8]], rows_in.at[b],
                    gsem[b]).wait()

                rows_cur = rows_in.at[b]
                tvec = jnp.broadcast_to(t, (16,))

                @plsc.parallel_loop(0, _D, unroll=2)
                def _d_body(d):
                    dvec = jnp.broadcast_to(d, (16,))
                    pv = plsc.load_gather(pos_v, [tvec, dvec])
                    for s in range(_BSUB // 16):
                        v = plsc.load_gather(rows_cur, [rowvec[s], dvec])
                        rows_out[so, d, pl.ds(16 * s, 16)] = v * _SCALE + pv

                for k in range(_D // 8):
                    pltpu.async_copy(
                        rows_out.at[so, pl.ds(8 * k, 8)],
                        out_hbm.at[(t * (_D // 8) + k) * _NBB + wid],
                        wsem[so])
                start_gather(jnp.minimum(t + _NIN, _T - 1), b)
            return carry

        lax.fori_loop(0, _T // _NIN, group_body, 0)

        for b in range(_NIN):
            pltpu.make_async_copy(
                table_hbm.at[idx_slab.at[0, 0]], rows_in.at[b],
                gsem[b]).wait()
        for so in range(_NOUT):
            for k in range(_D // 8):
                pltpu.make_async_copy(
                    rows_out.at[so, pl.ds(8 * k, 8)], out_hbm.at[k],
                    wsem[so]).wait()

    return body


@jax.jit
def kernel(x, embedding_table):
    b, t = x.shape
    xP = (x.astype(jnp.int32).T
          .reshape(_NTB, 8, _NBB, _BSUB).transpose(0, 2, 1, 3)
          .reshape(_NTB * _NBB, 8, _BSUB))
    pos = jnp.asarray(_pos_encoding())
    outP = _sc_kernel()(xP, embedding_table, pos)
    return (outP.reshape(_T, _D // 8, _NBB, 8, _BSUB)
            .transpose(2, 4, 0, 1, 3).reshape(b, t, _D))


# speedup vs baseline: 1.5592x; 1.0578x over previous
import functools

import jax
import jax.numpy as jnp
import numpy as np
from jax import lax
from jax.experimental import pallas as pl
from jax.experimental.pallas import tpu as pltpu
from jax.experimental.pallas import tpu_sc as plsc

_VOCAB = 1000000
_D = 64
_T = 200
_SCALE = 8.0

_NC = 2
_NS = 16
_NW = _NC * _NS
_CHUNK = 128
_NBUF = 4


def _pos_encoding_ext() -> np.ndarray:
    depth = _D / 2
    positions = np.arange(_T)[:, np.newaxis]
    depths = np.arange(depth)[np.newaxis, :] / depth
    angle_rads = positions * (1.0 / 10000**depths)
    pos = np.concatenate([np.sin(angle_rads), np.cos(angle_rads)], axis=-1)
    pos = pos.astype(np.float32)
    return np.concatenate([pos, pos[:_CHUNK]], axis=0)


def _sc_kernel(n_rows: int):
    rows_per_w = n_rows // _NW
    n_chunks = rows_per_w // _CHUNK
    n_groups = n_chunks // _NBUF
    mesh = plsc.VectorSubcoreMesh(
        core_axis_name="c", subcore_axis_name="s", num_cores=_NC,
        num_subcores=_NS)

    @functools.partial(
        pl.kernel,
        out_type=jax.ShapeDtypeStruct((n_rows, _D), jnp.float32),
        mesh=mesh,
        scratch_types=[
            pltpu.VMEM((n_chunks, _CHUNK), jnp.int32),
            pltpu.VMEM((_NBUF, _CHUNK, _D), jnp.float32),
            pltpu.VMEM((_NBUF, _CHUNK, _D), jnp.float32),
            pltpu.VMEM((_T + _CHUNK, _D), jnp.float32),
        ] + [pltpu.SemaphoreType.DMA] * (2 * _NBUF),
        compiler_params=pltpu.CompilerParams(
            use_tc_tiling_on_sc=False, skip_device_barrier=True),
    )
    def body(x_hbm, table_hbm, pos_hbm, out_hbm, idx_slab, rows_in, rows_out,
             pos_v, *sems):
        gsem = sems[:_NBUF]
        wsem = sems[_NBUF:]
        wid = lax.axis_index("s") * _NC + lax.axis_index("c")
        pltpu.sync_copy(pos_hbm, pos_v)
        pltpu.sync_copy(x_hbm.at[pl.ds(wid * n_chunks, n_chunks)], idx_slab)
        w_base = wid * rows_per_w

        def start_gather(g, b):
            pltpu.async_copy(
                table_hbm.at[idx_slab.at[g]], rows_in.at[b], gsem[b])

        for b in range(_NBUF):
            start_gather(b, b)

        def group_body(gi, carry):
            g0 = gi * _NBUF
            for b in range(_NBUF):
                g = g0 + b
                base = w_base + g * _CHUNK

                @pl.when(gi > 0)
                def _wait_wb():
                    pltpu.make_async_copy(
                        rows_out.at[b], out_hbm.at[pl.ds(0, _CHUNK)],
                        wsem[b]).wait()

                pltpu.make_async_copy(
                    table_hbm.at[idx_slab.at[g]], rows_in.at[b],
                    gsem[b]).wait()

                p0 = lax.rem(g * _CHUNK, _T)

                @plsc.parallel_loop(0, _CHUNK, unroll=4)
                def _row_body(i):
                    for j in range(_D // 16):
                        v = rows_in[b, i, pl.ds(j * 16, 16)]
                        p = pos_v[p0 + i, pl.ds(j * 16, 16)]
                        rows_out[b, i, pl.ds(j * 16, 16)] = v * _SCALE + p

                start_gather(jnp.minimum(g + _NBUF, n_chunks - 1), b)
                pltpu.async_copy(
                    rows_out.at[b], out_hbm.at[pl.ds(base, _CHUNK)], wsem[b])
            return carry

        lax.fori_loop(0, n_groups, group_body, 0)

        for b in range(_NBUF):
            pltpu.make_async_copy(
                table_hbm.at[idx_slab.at[0]], rows_in.at[b], gsem[b]).wait()
            pltpu.make_async_copy(
                rows_out.at[b], out_hbm.at[pl.ds(0, _CHUNK)], wsem[b]).wait()

    return body


@jax.jit
def kernel(x, embedding_table):
    b, t = x.shape
    x_flat = x.reshape(-1, _CHUNK).astype(jnp.int32)
    pos_ext = jnp.asarray(_pos_encoding_ext())
    out = _sc_kernel(b * t)(x_flat, embedding_table, pos_ext)
    return out.reshape(b, t, _D)


# speedup vs baseline: 1.6439x; 1.0544x over previous
import functools

import jax
import jax.numpy as jnp
import numpy as np
from jax import lax
from jax.experimental import pallas as pl
from jax.experimental.pallas import tpu as pltpu
from jax.experimental.pallas import tpu_sc as plsc

_VOCAB = 1000000
_D = 64
_T = 200
_B = 4096
_SCALE = 8.0

_NC = 2
_NS = 16
_NW = _NC * _NS
_BSUB = 128
_NBB = _B // _BSUB
_NTB = _T // 8
_NIN = 4
_NOUT = 2


def _pos_encoding() -> np.ndarray:
    depth = _D / 2
    positions = np.arange(_T)[:, np.newaxis]
    depths = np.arange(depth)[np.newaxis, :] / depth
    angle_rads = positions * (1.0 / 10000**depths)
    pos = np.concatenate([np.sin(angle_rads), np.cos(angle_rads)], axis=-1)
    return pos.astype(np.float32)


def _sc_kernel():
    mesh = plsc.VectorSubcoreMesh(
        core_axis_name="c", subcore_axis_name="s", num_cores=_NC,
        num_subcores=_NS)

    @functools.partial(
        pl.kernel,
        out_type=jax.ShapeDtypeStruct((_T * (_D // 8) * _NBB, 8, _BSUB),
                                      jnp.float32),
        mesh=mesh,
        scratch_types=[
            pltpu.VMEM((_NTB, 8, _BSUB), jnp.int32),
            pltpu.VMEM((_NIN, _BSUB, _D), jnp.float32),
            pltpu.VMEM((_NOUT, _D, _BSUB), jnp.float32),
            pltpu.VMEM((_T, _D), jnp.float32),
        ] + [pltpu.SemaphoreType.DMA] * (_NIN + _NOUT),
        compiler_params=pltpu.CompilerParams(
            use_tc_tiling_on_sc=False, needs_layout_passes=False),
    )
    def body(x_hbm, table_hbm, pos_hbm, out_hbm, idx_slab, rows_in, rows_out,
             pos_v, *sems):
        gsem = sems[:_NIN]
        wsem = sems[_NIN:]
        wid = lax.axis_index("s") * _NC + lax.axis_index("c")
        pltpu.sync_copy(pos_hbm, pos_v)
        for tb in range(_NTB):
            pltpu.sync_copy(x_hbm.at[tb * _NBB + wid], idx_slab.at[tb])

        def start_gather(t, b):
            pltpu.async_copy(
                table_hbm.at[idx_slab.at[t // 8, t % 8]], rows_in.at[b],
                gsem[b])

        for b in range(_NIN):
            start_gather(b, b)

        lane = lax.iota(jnp.int32, 16)
        rowvec = [lane + 16 * s for s in range(_BSUB // 16)]

        def group_body(gi, carry):
            t0 = gi * _NIN
            for b in range(_NIN):
                t = t0 + b
                so = b % _NOUT

                def _wait_wb():
                    for k in range(_D // 8):
                        pltpu.make_async_copy(
                            rows_out.at[so, pl.ds(8 * k, 8)], out_hbm.at[k],
                            wsem[so]).wait()

                if b >= _NOUT:
                    _wait_wb()
                else:
                    pl.when(gi > 0)(_wait_wb)

                pltpu.make_async_copy(
                    table_hbm.at[idx_slab.at[t // 8, t % 8]], rows_in.at[b],
                    gsem[b]).wait()

                rows_cur = rows_in.at[b]
                out_cur = rows_out.at[so]
                tvec = jnp.broadcast_to(t, (16,))

                @plsc.parallel_loop(0, 16, unroll=2)
                def _r_body(r):
                    bc = (lane + r) & 15
                    for s in range(_BSUB // 16):
                        for j in range(_D // 16):
                            colv = bc + 16 * j
                            v = plsc.load_gather(rows_cur, [rowvec[s], colv])
                            p = plsc.load_gather(pos_v, [tvec, colv])
                            plsc.store_scatter(
                                out_cur, [colv, rowvec[s]], v * _SCALE + p)

                for k in range(_D // 8):
                    pltpu.async_copy(
                        rows_out.at[so, pl.ds(8 * k, 8)],
                        out_hbm.at[(t * (_D // 8) + k) * _NBB + wid],
                        wsem[so])
                start_gather(jnp.minimum(t + _NIN, _T - 1), b)
            return carry

        lax.fori_loop(0, _T // _NIN, group_body, 0)

        for b in range(_NIN):
            pltpu.make_async_copy(
                table_hbm.at[idx_slab.at[0, 0]], rows_in.at[b],
                gsem[b]).wait()
        for so in range(_NOUT):
            for k in range(_D // 8):
                pltpu.make_async_copy(
                    rows_out.at[so, pl.ds(8 * k, 8)], out_hbm.at[k],
                    wsem[so]).wait()

    return body


@jax.jit
def kernel(x, embedding_table):
    b, t = x.shape
    xP = (x.astype(jnp.int32).T
          .reshape(_NTB, 8, _NBB, _BSUB).transpose(0, 2, 1, 3)
          .reshape(_NTB * _NBB, 8, _BSUB))
    pos = jnp.asarray(_pos_encoding())
    outP = _sc_kernel()(xP, embedding_table, pos)
    return (outP.reshape(_T, _D // 8, _NBB, 8, _BSUB)
            .transpose(2, 4, 0, 1, 3).reshape(b, t, _D))


# speedup vs baseline: 2.1308x; 1.2962x over previous
import functools

import jax
import jax.numpy as jnp
import numpy as np
from jax import lax
from jax.experimental import pallas as pl
from jax.experimental.pallas import tpu as pltpu
from jax.experimental.pallas import tpu_sc as plsc

_VOCAB = 1000000
_D = 64
_T = 200
_B = 4096
_SCALE = 8.0

_NC = 2
_NS = 16
_NW = _NC * _NS
_BSUB = 128
_NBB = _B // _BSUB
_NTB = _T // 8
_NIN = 4
_NOUT = 2


def _pos_encoding() -> np.ndarray:
    depth = _D / 2
    positions = np.arange(_T)[:, np.newaxis]
    depths = np.arange(depth)[np.newaxis, :] / depth
    angle_rads = positions * (1.0 / 10000**depths)
    pos = np.concatenate([np.sin(angle_rads), np.cos(angle_rads)], axis=-1)
    return pos.astype(np.float32)


def _sc_kernel():
    mesh = plsc.VectorSubcoreMesh(
        core_axis_name="c", subcore_axis_name="s", num_cores=_NC,
        num_subcores=_NS)

    @functools.partial(
        pl.kernel,
        out_type=jax.ShapeDtypeStruct((_T * (_D // 8) * _NBB, 8, _BSUB),
                                      jnp.float32),
        mesh=mesh,
        scratch_types=[
            pltpu.VMEM((_NTB, 8, _BSUB), jnp.int32),
            pltpu.VMEM((_NIN, _BSUB, _D), jnp.float32),
            pltpu.VMEM((_NOUT, _D, _BSUB), jnp.float32),
            pltpu.VMEM((_T, _D), jnp.float32),
        ] + [pltpu.SemaphoreType.DMA] * (_NIN + _NOUT),
        compiler_params=pltpu.CompilerParams(
            use_tc_tiling_on_sc=False, needs_layout_passes=False),
    )
    def body(x_hbm, table_hbm, pos_hbm, out_hbm, idx_slab, rows_in, rows_out,
             pos_v, *sems):
        gsem = sems[:_NIN]
        wsem = sems[_NIN:]
        wid = lax.axis_index("s") * _NC + lax.axis_index("c")
        pltpu.sync_copy(pos_hbm, pos_v)
        for tb in range(_NTB):
            pltpu.sync_copy(x_hbm.at[tb * _NBB + wid], idx_slab.at[tb])

        def start_gather(t, b):
            pltpu.async_copy(
                table_hbm.at[idx_slab.at[t // 8, t % 8]], rows_in.at[b],
                gsem[b])

        for b in range(_NIN):
            start_gather(b, b)

        lane = lax.iota(jnp.int32, 16)
        rowvec = [lane + 16 * s for s in range(_BSUB // 16)]

        def group_body(gi, carry):
            t0 = gi * _NIN
            for b in range(_NIN):
                t = t0 + b
                so = b % _NOUT

                def _wait_wb():
                    for k in range(_D // 8):
                        pltpu.make_async_copy(
                            rows_out.at[so, pl.ds(8 * k, 8)], out_hbm.at[k],
                            wsem[so]).wait()

                if b >= _NOUT:
                    _wait_wb()
                else:
                    pl.when(gi > 0)(_wait_wb)

                pltpu.make_async_copy(
                    table_hbm.at[idx_slab.at[t // 8, t % 8]], rows_in.at[b],
                    gsem[b]).wait()

                rows_cur = rows_in.at[b]
                out_cur = rows_out.at[so]
                tvec = jnp.broadcast_to(t, (16,))

                @plsc.parallel_loop(0, 16, unroll=2)
                def _r_body(r):
                    bc = (lane + r) & 15
                    for j in range(_D // 16):
                        colv = bc + 16 * j
                        p = plsc.load_gather(pos_v, [tvec, colv])
                        for s in range(_BSUB // 16):
                            v = plsc.load_gather(rows_cur, [rowvec[s], colv])
                            plsc.store_scatter(
                                out_cur, [colv, rowvec[s]], v * _SCALE + p)

                for k in range(_D // 8):
                    pltpu.async_copy(
                        rows_out.at[so, pl.ds(8 * k, 8)],
                        out_hbm.at[(t * (_D // 8) + k) * _NBB + wid],
                        wsem[so])
                start_gather(jnp.minimum(t + _NIN, _T - 1), b)
            return carry

        lax.fori_loop(0, _T // _NIN, group_body, 0)

        for b in range(_NIN):
            pltpu.make_async_copy(
                table_hbm.at[idx_slab.at[0, 0]], rows_in.at[b],
                gsem[b]).wait()
        for so in range(_NOUT):
            for k in range(_D // 8):
                pltpu.make_async_copy(
                    rows_out.at[so, pl.ds(8 * k, 8)], out_hbm.at[k],
                    wsem[so]).wait()

    return body


@jax.jit
def kernel(x, embedding_table):
    b, t = x.shape
    xP = (x.astype(jnp.int32).T
          .reshape(_NTB, 8, _NBB, _BSUB).transpose(0, 2, 1, 3)
          .reshape(_NTB * _NBB, 8, _BSUB))
    pos = jnp.asarray(_pos_encoding())
    outP = _sc_kernel()(xP, embedding_table, pos)
    return (outP.reshape(_T, _D // 8, _NBB, 8, _BSUB)
            .transpose(2, 4, 0, 1, 3).reshape(b, t, _D))
